# baseline (speedup 1.0000x reference)
import functools

import jax
import jax.numpy as jnp
from jax import lax
from jax.experimental import pallas as pl
from jax.experimental.pallas import tpu as pltpu
from jax.experimental.pallas import tpu_sc as plsc

N_NODES_C = 10000
N_EDGES_C = 320000
D_C = 128

NC = 2
NS = 16
NW = NC * NS

CH = 128
NCHUNK0 = 100
NCHUNK1 = 58
NCHUNK = max(NCHUNK0, NCHUNK1)
E_PAD = (NCHUNK0 + NCHUNK1) * NS * CH
ACC_ROWS = 10240
ROWS_PER_TILE = ACC_ROWS // NS


def _sc_body(src_hbm, dst_hbm, hn_hbm, zeros_hbm, out_hbm,
             src_v, dst_v, rows_v, acc, sg0):
    c = lax.axis_index("c")
    s = lax.axis_index("s")
    wid = c * NS + s

    row0 = s * ROWS_PER_TILE
    pltpu.sync_copy(zeros_hbm.at[pl.ds(row0, ROWS_PER_TILE)],
                    acc.at[pl.ds(row0, ROWS_PER_TILE)])
    pltpu.sync_copy(src_hbm.at[wid], src_v)
    pltpu.sync_copy(dst_hbm.at[wid], dst_v)
    plsc.subcore_barrier()

    def chunk(j, carry):
        pltpu.async_copy(hn_hbm.at[src_v.at[j]], rows_v, sg0).wait()
        pltpu.sync_copy(rows_v, acc.at[dst_v.at[j]], add=True)
        return carry

    @pl.when(c == 0)
    def _():
        lax.fori_loop(0, NCHUNK0, chunk, 0, unroll=False)

    @pl.when(c == 1)
    def _():
        lax.fori_loop(0, NCHUNK1, chunk, 0, unroll=False)

    plsc.subcore_barrier()
    pltpu.sync_copy(acc.at[pl.ds(row0, ROWS_PER_TILE)],
                    out_hbm.at[c, pl.ds(row0, ROWS_PER_TILE)])


@functools.cache
def _sc_aggregate():
    return functools.partial(
        pl.kernel,
        out_type=jax.ShapeDtypeStruct((NC, ACC_ROWS, D_C), jnp.float32),
        mesh=plsc.VectorSubcoreMesh(core_axis_name="c", subcore_axis_name="s",
                                    num_cores=NC, num_subcores=NS),
        scratch_types=[
            pltpu.VMEM((NCHUNK, CH), jnp.int32),
            pltpu.VMEM((NCHUNK, CH), jnp.int32),
            pltpu.VMEM((CH, D_C), jnp.float32),
            pltpu.VMEM_SHARED((ACC_ROWS, D_C), jnp.float32),
            pltpu.SemaphoreType.DMA,
        ],
    )(_sc_body)


def _mlp_body(hn_ref, p_ref, w1a_ref, w1b_ref, b1_ref, w2_ref, b2_ref,
              w3_ref, b3_ref, out_ref):
    aggr = p_ref[0] + p_ref[1]
    hi = lax.Precision.DEFAULT
    h = jnp.dot(hn_ref[...], w1a_ref[...], precision=hi,
                preferred_element_type=jnp.float32)
    h += jnp.dot(aggr, w1b_ref[...], precision=hi,
                 preferred_element_type=jnp.float32)
    h = jnp.maximum(h + b1_ref[...], 0.0)
    h = jnp.maximum(
        jnp.dot(h, w2_ref[...], precision=hi,
                preferred_element_type=jnp.float32) + b2_ref[...], 0.0)
    out_ref[...] = jnp.dot(h, w3_ref[...], precision=hi,
                           preferred_element_type=jnp.float32) + b3_ref[...]


def _mlp(hn, partials, w1a, w1b, b1, w2, b2, w3, b3):
    blk = 2000
    grid = (N_NODES_C // blk,)
    wspec = pl.BlockSpec((D_C, D_C), lambda i: (0, 0))
    bspec = pl.BlockSpec((1, D_C), lambda i: (0, 0))
    return pl.pallas_call(
        _mlp_body,
        grid=grid,
        in_specs=[
            pl.BlockSpec((blk, D_C), lambda i: (i, 0)),
            pl.BlockSpec((NC, blk, D_C), lambda i: (0, i, 0)),
            wspec, wspec, bspec, wspec, bspec, wspec, bspec,
        ],
        out_specs=pl.BlockSpec((blk, D_C), lambda i: (i, 0)),
        out_shape=jax.ShapeDtypeStruct((N_NODES_C, D_C), jnp.float32),
    )(hn, partials, w1a, w1b, b1, w2, b2, w3, b3)


def kernel(hn, edge_index, he, W1, b1, W2, b2, W3, b3):
    del he
    src = edge_index[0]
    dst = edge_index[1]
    pad = E_PAD - N_EDGES_C
    src = jnp.concatenate([src, jnp.zeros((pad,), jnp.int32)])
    dst = jnp.concatenate([dst, jnp.full((pad,), N_NODES_C, jnp.int32)])
    nA = NS * NCHUNK0 * CH

    def slab(flat, fill):
        def part(x, n):
            x = x.reshape(NS, n, CH)
            if n < NCHUNK:
                x = jnp.concatenate(
                    [x, jnp.full((NS, NCHUNK - n, CH), fill, jnp.int32)],
                    axis=1)
            return x

        return jnp.concatenate(
            [part(flat[:nA], NCHUNK0), part(flat[nA:], NCHUNK1)], axis=0)

    src = slab(src, 0)
    dst = slab(dst, N_NODES_C)
    zeros = jnp.zeros((ACC_ROWS, D_C), jnp.float32)

    partials = _sc_aggregate()(src, dst, hn, zeros)

    w1a = W1[:, :D_C].T
    w1b = W1[:, D_C:].T
    return _mlp(hn, partials, w1a, w1b, b1.reshape(1, -1),
                W2.T, b2.reshape(1, -1), W3.T, b3.reshape(1, -1))


# speedup vs baseline: 1.0266x; 1.0266x over previous
import functools

import jax
import jax.numpy as jnp
from jax import lax
from jax.experimental import pallas as pl
from jax.experimental.pallas import tpu as pltpu
from jax.experimental.pallas import tpu_sc as plsc

N_NODES_C = 10000
N_EDGES_C = 320000
D_C = 128

NC = 2
NS = 16
NW = NC * NS

CH = 128
NCHUNK0 = 106
NCHUNK1 = 52
NCHUNK = max(NCHUNK0, NCHUNK1)
E_PAD = (NCHUNK0 + NCHUNK1) * NS * CH
ACC_ROWS = 10240
ROWS_PER_TILE = ACC_ROWS // NS


def _sc_body(src_hbm, dst_hbm, hn_hbm, zeros_hbm, out_hbm,
             src_v, dst_v, rows_v, acc, sg0):
    c = lax.axis_index("c")
    s = lax.axis_index("s")
    wid = c * NS + s

    row0 = s * ROWS_PER_TILE
    pltpu.sync_copy(zeros_hbm.at[pl.ds(row0, ROWS_PER_TILE)],
                    acc.at[pl.ds(row0, ROWS_PER_TILE)])
    pltpu.sync_copy(src_hbm.at[wid], src_v)
    pltpu.sync_copy(dst_hbm.at[wid], dst_v)
    plsc.subcore_barrier()

    def chunk(j, carry):
        pltpu.async_copy(hn_hbm.at[src_v.at[j]], rows_v, sg0).wait()
        pltpu.sync_copy(rows_v, acc.at[dst_v.at[j]], add=True)
        return carry

    @pl.when(c == 0)
    def _():
        lax.fori_loop(0, NCHUNK0, chunk, 0, unroll=False)

    @pl.when(c == 1)
    def _():
        lax.fori_loop(0, NCHUNK1, chunk, 0, unroll=False)

    plsc.subcore_barrier()
    pltpu.sync_copy(acc.at[pl.ds(row0, ROWS_PER_TILE)],
                    out_hbm.at[c, pl.ds(row0, ROWS_PER_TILE)])


@functools.cache
def _sc_aggregate():
    return functools.partial(
        pl.kernel,
        out_type=jax.ShapeDtypeStruct((NC, ACC_ROWS, D_C), jnp.float32),
        mesh=plsc.VectorSubcoreMesh(core_axis_name="c", subcore_axis_name="s",
                                    num_cores=NC, num_subcores=NS),
        scratch_types=[
            pltpu.VMEM((NCHUNK, CH), jnp.int32),
            pltpu.VMEM((NCHUNK, CH), jnp.int32),
            pltpu.VMEM((CH, D_C), jnp.float32),
            pltpu.VMEM_SHARED((ACC_ROWS, D_C), jnp.float32),
            pltpu.SemaphoreType.DMA,
        ],
    )(_sc_body)


def _mlp_body(hn_ref, p_ref, w1a_ref, w1b_ref, b1_ref, w2_ref, b2_ref,
              w3_ref, b3_ref, out_ref):
    aggr = p_ref[0] + p_ref[1]
    hi = lax.Precision.DEFAULT
    h = jnp.dot(hn_ref[...], w1a_ref[...], precision=hi,
                preferred_element_type=jnp.float32)
    h += jnp.dot(aggr, w1b_ref[...], precision=hi,
                 preferred_element_type=jnp.float32)
    h = jnp.maximum(h + b1_ref[...], 0.0)
    h = jnp.maximum(
        jnp.dot(h, w2_ref[...], precision=hi,
                preferred_element_type=jnp.float32) + b2_ref[...], 0.0)
    out_ref[...] = jnp.dot(h, w3_ref[...], precision=hi,
                           preferred_element_type=jnp.float32) + b3_ref[...]


def _mlp(hn, partials, w1a, w1b, b1, w2, b2, w3, b3):
    blk = 2000
    grid = (N_NODES_C // blk,)
    wspec = pl.BlockSpec((D_C, D_C), lambda i: (0, 0))
    bspec = pl.BlockSpec((1, D_C), lambda i: (0, 0))
    return pl.pallas_call(
        _mlp_body,
        grid=grid,
        in_specs=[
            pl.BlockSpec((blk, D_C), lambda i: (i, 0)),
            pl.BlockSpec((NC, blk, D_C), lambda i: (0, i, 0)),
            wspec, wspec, bspec, wspec, bspec, wspec, bspec,
        ],
        out_specs=pl.BlockSpec((blk, D_C), lambda i: (i, 0)),
        out_shape=jax.ShapeDtypeStruct((N_NODES_C, D_C), jnp.float32),
    )(hn, partials, w1a, w1b, b1, w2, b2, w3, b3)


def kernel(hn, edge_index, he, W1, b1, W2, b2, W3, b3):
    del he
    src = edge_index[0]
    dst = edge_index[1]
    pad = E_PAD - N_EDGES_C
    src = jnp.concatenate([src, jnp.zeros((pad,), jnp.int32)])
    dst = jnp.concatenate([dst, jnp.full((pad,), N_NODES_C, jnp.int32)])
    nA = NS * NCHUNK0 * CH

    def slab(flat, fill):
        def part(x, n):
            x = x.reshape(NS, n, CH)
            if n < NCHUNK:
                x = jnp.concatenate(
                    [x, jnp.full((NS, NCHUNK - n, CH), fill, jnp.int32)],
                    axis=1)
            return x

        return jnp.concatenate(
            [part(flat[:nA], NCHUNK0), part(flat[nA:], NCHUNK1)], axis=0)

    src = slab(src, 0)
    dst = slab(dst, N_NODES_C)
    zeros = jnp.zeros((ACC_ROWS, D_C), jnp.float32)

    partials = _sc_aggregate()(src, dst, hn, zeros)

    w1a = W1[:, :D_C].T
    w1b = W1[:, D_C:].T
    return _mlp(hn, partials, w1a, w1b, b1.reshape(1, -1),
                W2.T, b2.reshape(1, -1), W3.T, b3.reshape(1, -1))


# speedup vs baseline: 1.0287x; 1.0021x over previous
import functools

import jax
import jax.numpy as jnp
from jax import lax
from jax.experimental import pallas as pl
from jax.experimental.pallas import tpu as pltpu
from jax.experimental.pallas import tpu_sc as plsc

N_NODES_C = 10000
N_EDGES_C = 320000
D_C = 128

NC = 2
NS = 16
NW = NC * NS

CH = 128
NCHUNK0 = 106
NCHUNK1 = 52
NCHUNK = max(NCHUNK0, NCHUNK1)
E_PAD = (NCHUNK0 + NCHUNK1) * NS * CH
ACC_ROWS = 10240
ROWS_PER_TILE = ACC_ROWS // NS


def _sc_body(src_hbm, dst_hbm, hn_hbm, zeros_hbm, out_hbm,
             src_v, dst_v, rows_v, acc, sg0):
    c = lax.axis_index("c")
    s = lax.axis_index("s")
    wid = c * NS + s

    row0 = s * ROWS_PER_TILE
    pltpu.sync_copy(zeros_hbm.at[pl.ds(row0, ROWS_PER_TILE)],
                    acc.at[pl.ds(row0, ROWS_PER_TILE)])
    pltpu.sync_copy(src_hbm.at[wid], src_v)
    pltpu.sync_copy(dst_hbm.at[wid], dst_v)
    plsc.subcore_barrier()

    def chunk(j, carry):
        pltpu.async_copy(hn_hbm.at[src_v.at[j]], rows_v, sg0).wait()
        pltpu.sync_copy(rows_v, acc.at[dst_v.at[j]], add=True)
        return carry

    @pl.when(c == 0)
    def _():
        lax.fori_loop(0, NCHUNK0, chunk, 0, unroll=False)

    @pl.when(c == 1)
    def _():
        lax.fori_loop(0, NCHUNK1, chunk, 0, unroll=False)

    plsc.subcore_barrier()
    pltpu.sync_copy(acc.at[pl.ds(row0, ROWS_PER_TILE)],
                    out_hbm.at[c, pl.ds(row0, ROWS_PER_TILE)])


@functools.cache
def _sc_aggregate():
    return functools.partial(
        pl.kernel,
        out_type=jax.ShapeDtypeStruct((NC, ACC_ROWS, D_C), jnp.float32),
        mesh=plsc.VectorSubcoreMesh(core_axis_name="c", subcore_axis_name="s",
                                    num_cores=NC, num_subcores=NS),
        scratch_types=[
            pltpu.VMEM((NCHUNK, CH), jnp.int32),
            pltpu.VMEM((NCHUNK, CH), jnp.int32),
            pltpu.VMEM((CH, D_C), jnp.float32),
            pltpu.VMEM_SHARED((ACC_ROWS, D_C), jnp.float32),
            pltpu.SemaphoreType.DMA,
        ],
    )(_sc_body)


def _mlp_body(hn_ref, p_ref, w1a_ref, w1b_ref, b1_ref, w2_ref, b2_ref,
              w3_ref, b3_ref, out_ref):
    aggr = p_ref[0] + p_ref[1]
    hi = lax.Precision.DEFAULT
    h = jnp.dot(hn_ref[...], w1a_ref[...], precision=hi,
                preferred_element_type=jnp.float32)
    h += jnp.dot(aggr, w1b_ref[...], precision=hi,
                 preferred_element_type=jnp.float32)
    h = jnp.maximum(h + b1_ref[...], 0.0)
    h = jnp.maximum(
        jnp.dot(h, w2_ref[...], precision=hi,
                preferred_element_type=jnp.float32) + b2_ref[...], 0.0)
    out_ref[...] = jnp.dot(h, w3_ref[...], precision=hi,
                           preferred_element_type=jnp.float32) + b3_ref[...]


def _mlp(hn, partials, w1a, w1b, b1, w2, b2, w3, b3):
    blk = 5000
    grid = (N_NODES_C // blk,)
    wspec = pl.BlockSpec((D_C, D_C), lambda i: (0, 0))
    bspec = pl.BlockSpec((1, D_C), lambda i: (0, 0))
    return pl.pallas_call(
        _mlp_body,
        grid=grid,
        in_specs=[
            pl.BlockSpec((blk, D_C), lambda i: (i, 0)),
            pl.BlockSpec((NC, blk, D_C), lambda i: (0, i, 0)),
            wspec, wspec, bspec, wspec, bspec, wspec, bspec,
        ],
        out_specs=pl.BlockSpec((blk, D_C), lambda i: (i, 0)),
        out_shape=jax.ShapeDtypeStruct((N_NODES_C, D_C), jnp.float32),
    )(hn, partials, w1a, w1b, b1, w2, b2, w3, b3)


def kernel(hn, edge_index, he, W1, b1, W2, b2, W3, b3):
    del he
    src = edge_index[0]
    dst = edge_index[1]
    pad = E_PAD - N_EDGES_C
    src = jnp.concatenate([src, jnp.zeros((pad,), jnp.int32)])
    dst = jnp.concatenate([dst, jnp.full((pad,), N_NODES_C, jnp.int32)])
    nA = NS * NCHUNK0 * CH

    def slab(flat, fill):
        def part(x, n):
            x = x.reshape(NS, n, CH)
            if n < NCHUNK:
                x = jnp.concatenate(
                    [x, jnp.full((NS, NCHUNK - n, CH), fill, jnp.int32)],
                    axis=1)
            return x

        return jnp.concatenate(
            [part(flat[:nA], NCHUNK0), part(flat[nA:], NCHUNK1)], axis=0)

    src = slab(src, 0)
    dst = slab(dst, N_NODES_C)
    zeros = jnp.zeros((ACC_ROWS, D_C), jnp.float32)

    partials = _sc_aggregate()(src, dst, hn, zeros)

    w1a = W1[:, :D_C].T
    w1b = W1[:, D_C:].T
    return _mlp(hn, partials, w1a, w1b, b1.reshape(1, -1),
                W2.T, b2.reshape(1, -1), W3.T, b3.reshape(1, -1))


# speedup vs baseline: 1.0850x; 1.0547x over previous
import functools

import jax
import jax.numpy as jnp
from jax import lax
from jax.experimental import pallas as pl
from jax.experimental.pallas import tpu as pltpu
from jax.experimental.pallas import tpu_sc as plsc

N_NODES_C = 10000
N_EDGES_C = 320000
D_C = 128

NC = 2
NS = 16
NW = NC * NS

CH = 128
NCHUNK0 = 112
NCHUNK1 = 46
NCHUNK = max(NCHUNK0, NCHUNK1)
E_PAD = (NCHUNK0 + NCHUNK1) * NS * CH
ACC_ROWS = 10240
ROWS_PER_TILE = ACC_ROWS // NS


def _sc_body(src_hbm, dst_hbm, hn_hbm, zeros_hbm, out_hbm,
             src_v, dst_v, rows_v, acc, sg0):
    c = lax.axis_index("c")
    s = lax.axis_index("s")
    wid = c * NS + s

    row0 = s * ROWS_PER_TILE
    pltpu.sync_copy(zeros_hbm.at[pl.ds(row0, ROWS_PER_TILE)],
                    acc.at[pl.ds(row0, ROWS_PER_TILE)])
    pltpu.sync_copy(src_hbm.at[wid], src_v)
    pltpu.sync_copy(dst_hbm.at[wid], dst_v)
    plsc.subcore_barrier()

    def chunk(j, carry):
        pltpu.async_copy(hn_hbm.at[src_v.at[j]], rows_v, sg0).wait()
        pltpu.sync_copy(rows_v, acc.at[dst_v.at[j]], add=True)
        return carry

    @pl.when(c == 0)
    def _():
        lax.fori_loop(0, NCHUNK0, chunk, 0, unroll=False)

    @pl.when(c == 1)
    def _():
        lax.fori_loop(0, NCHUNK1, chunk, 0, unroll=False)

    plsc.subcore_barrier()
    pltpu.sync_copy(acc.at[pl.ds(row0, ROWS_PER_TILE)],
                    out_hbm.at[c, pl.ds(row0, ROWS_PER_TILE)])


@functools.cache
def _sc_aggregate():
    return functools.partial(
        pl.kernel,
        out_type=jax.ShapeDtypeStruct((NC, ACC_ROWS, D_C), jnp.float32),
        mesh=plsc.VectorSubcoreMesh(core_axis_name="c", subcore_axis_name="s",
                                    num_cores=NC, num_subcores=NS),
        scratch_types=[
            pltpu.VMEM((NCHUNK, CH), jnp.int32),
            pltpu.VMEM((NCHUNK, CH), jnp.int32),
            pltpu.VMEM((CH, D_C), jnp.float32),
            pltpu.VMEM_SHARED((ACC_ROWS, D_C), jnp.float32),
            pltpu.SemaphoreType.DMA,
        ],
    )(_sc_body)


def _mlp_body(hn_ref, p_ref, w1a_ref, w1b_ref, b1_ref, w2_ref, b2_ref,
              w3_ref, b3_ref, out_ref):
    aggr = p_ref[0] + p_ref[1]
    hi = lax.Precision.DEFAULT
    h = jnp.dot(hn_ref[...], w1a_ref[...], precision=hi,
                preferred_element_type=jnp.float32)
    h += jnp.dot(aggr, w1b_ref[...], precision=hi,
                 preferred_element_type=jnp.float32)
    h = jnp.maximum(h + b1_ref[...], 0.0)
    h = jnp.maximum(
        jnp.dot(h, w2_ref[...], precision=hi,
                preferred_element_type=jnp.float32) + b2_ref[...], 0.0)
    out_ref[...] = jnp.dot(h, w3_ref[...], precision=hi,
                           preferred_element_type=jnp.float32) + b3_ref[...]


def _mlp(hn, partials, w1a, w1b, b1, w2, b2, w3, b3):
    blk = 5000
    grid = (N_NODES_C // blk,)
    wspec = pl.BlockSpec((D_C, D_C), lambda i: (0, 0))
    bspec = pl.BlockSpec((1, D_C), lambda i: (0, 0))
    return pl.pallas_call(
        _mlp_body,
        grid=grid,
        in_specs=[
            pl.BlockSpec((blk, D_C), lambda i: (i, 0)),
            pl.BlockSpec((NC, blk, D_C), lambda i: (0, i, 0)),
            wspec, wspec, bspec, wspec, bspec, wspec, bspec,
        ],
        out_specs=pl.BlockSpec((blk, D_C), lambda i: (i, 0)),
        out_shape=jax.ShapeDtypeStruct((N_NODES_C, D_C), jnp.float32),
    )(hn, partials, w1a, w1b, b1, w2, b2, w3, b3)


def kernel(hn, edge_index, he, W1, b1, W2, b2, W3, b3):
    del he
    src = edge_index[0]
    dst = edge_index[1]
    pad = E_PAD - N_EDGES_C
    src = jnp.concatenate([src, jnp.zeros((pad,), jnp.int32)])
    dst = jnp.concatenate([dst, jnp.full((pad,), N_NODES_C, jnp.int32)])
    nA = NS * NCHUNK0 * CH

    def slab(flat, fill):
        def part(x, n):
            x = x.reshape(NS, n, CH)
            if n < NCHUNK:
                x = jnp.concatenate(
                    [x, jnp.full((NS, NCHUNK - n, CH), fill, jnp.int32)],
                    axis=1)
            return x

        return jnp.concatenate(
            [part(flat[:nA], NCHUNK0), part(flat[nA:], NCHUNK1)], axis=0)

    src = slab(src, 0)
    dst = slab(dst, N_NODES_C)
    zeros = jnp.zeros((ACC_ROWS, D_C), jnp.float32)

    partials = _sc_aggregate()(src, dst, hn, zeros)

    w1a = W1[:, :D_C].T
    w1b = W1[:, D_C:].T
    return _mlp(hn, partials, w1a, w1b, b1.reshape(1, -1),
                W2.T, b2.reshape(1, -1), W3.T, b3.reshape(1, -1))
